# Initial kernel scaffold; baseline (speedup 1.0000x reference)
import jax
import jax.numpy as jnp
from jax.experimental import pallas as pl


def kernel(seq_len, relative_bias):
    raise NotImplementedError("write your pallas kernel here")


# speedup vs baseline: 59.1045x; 59.1045x over previous
import jax
import jax.numpy as jnp
from jax.experimental import pallas as pl
from jax.experimental.pallas import tpu as pltpu

NH = 16
MAXD = 32
S = 2048
W = 2 * MAXD + 1
MLEN = 2 * S
B = 256


def _body(table_ref, out_ref, m_ref):
    h = pl.program_id(0)
    b = pl.program_id(1)

    @pl.when(b == 0)
    def _build_master():
        t = jax.lax.broadcasted_iota(jnp.int32, (1, MLEN), 1)
        idx = jnp.clip(t - (S - 1), -MAXD, MAXD) + MAXD
        acc = jnp.full((1, MLEN), table_ref[0, 0, 0], dtype=jnp.float32)
        for k in range(1, W):
            acc = jnp.where(idx == k, table_ref[0, 0, k], acc)
        bm = jnp.broadcast_to(acc, (B, MLEN))
        m_ref[...] = pltpu.roll(bm, 0, 1, stride=1, stride_axis=0)

    base = (b * B - (S - 1)) % MLEN
    rolled = pltpu.roll(m_ref[...], base, 1)
    out_ref[0] = rolled[:, :S]


def kernel(seq_len, relative_bias):
    del seq_len
    out = pl.pallas_call(
        _body,
        grid=(NH, S // B),
        in_specs=[
            pl.BlockSpec((1, 1, W), lambda h, b: (h, 0, 0),
                         memory_space=pltpu.SMEM),
        ],
        out_specs=pl.BlockSpec((1, B, S), lambda h, b: (h, b, 0)),
        out_shape=jax.ShapeDtypeStruct((NH, S, S), jnp.float32),
        scratch_shapes=[pltpu.VMEM((B, MLEN), jnp.float32)],
    )(relative_bias.reshape(NH, 1, W))
    return out


# speedup vs baseline: 147.2521x; 2.4914x over previous
import jax
import jax.numpy as jnp
from jax.experimental import pallas as pl
from jax.experimental.pallas import tpu as pltpu

NH = 16
MAXD = 32
S = 2048
W = 2 * MAXD + 1
MLEN = 2 * S
B = 256
NB = S // B


def _body(table_ref, out_ref, m_ref):
    b = pl.program_id(1)

    @pl.when(b == 0)
    def _build_master():
        t = jax.lax.broadcasted_iota(jnp.int32, (1, MLEN), 1)
        idx = jnp.clip(t - S, -MAXD, MAXD) + MAXD
        acc = jnp.full((1, MLEN), table_ref[0, 0, 0], dtype=jnp.float32)
        for k in range(1, W):
            acc = jnp.where(idx == k, table_ref[0, 0, k], acc)
        bm = jnp.broadcast_to(acc, (B, MLEN))
        m_ref[...] = pltpu.roll(bm, 0, 1, stride=1, stride_axis=0)

    for bb in range(NB):
        @pl.when(b == bb)
        def _copy(bb=bb):
            c = S - B * bb
            out_ref[0] = m_ref[:, c:c + S]


def kernel(seq_len, relative_bias):
    del seq_len
    out = pl.pallas_call(
        _body,
        grid=(NH, NB),
        in_specs=[
            pl.BlockSpec((1, 1, W), lambda h, b: (h, 0, 0),
                         memory_space=pltpu.SMEM),
        ],
        out_specs=pl.BlockSpec((1, B, S), lambda h, b: (h, b, 0)),
        out_shape=jax.ShapeDtypeStruct((NH, S, S), jnp.float32),
        scratch_shapes=[pltpu.VMEM((B, MLEN), jnp.float32)],
    )(relative_bias.reshape(NH, 1, W))
    return out


# speedup vs baseline: 194.5909x; 1.3215x over previous
import jax
import jax.numpy as jnp
from jax.experimental import pallas as pl
from jax.experimental.pallas import tpu as pltpu

NH = 16
MAXD = 32
S = 2048
W = 2 * MAXD + 1
MLEN = 2 * S
B = 256
NB = S // B


def _copies(h, slot, m_ref, out_ref, sem):
    for bb in range(NB):
        c = S - B * bb
        yield pltpu.make_async_copy(
            m_ref.at[slot, :, pl.ds(c, S)],
            out_ref.at[h, pl.ds(bb * B, B), :],
            sem.at[slot],
        )


def _body(table_ref, out_ref, m_ref, sem):
    h = pl.program_id(0)
    slot = jax.lax.rem(h, 2)

    @pl.when(h >= 2)
    def _drain_prev():
        for cp in _copies(h - 2, slot, m_ref, out_ref, sem):
            cp.wait()

    t = jax.lax.broadcasted_iota(jnp.int32, (1, MLEN), 1)
    idx = jnp.clip(t - S, -MAXD, MAXD) + MAXD
    acc = jnp.full((1, MLEN), table_ref[0, 0, 0], dtype=jnp.float32)
    for k in range(1, W):
        acc = jnp.where(idx == k, table_ref[0, 0, k], acc)
    bm = jnp.broadcast_to(acc, (B, MLEN))
    m_ref[slot] = pltpu.roll(bm, 0, 1, stride=1, stride_axis=0)

    for cp in _copies(h, slot, m_ref, out_ref, sem):
        cp.start()

    @pl.when(h == NH - 1)
    def _drain_tail():
        for cp in _copies(h - 1, 1 - slot, m_ref, out_ref, sem):
            cp.wait()
        for cp in _copies(h, slot, m_ref, out_ref, sem):
            cp.wait()


def kernel(seq_len, relative_bias):
    del seq_len
    out = pl.pallas_call(
        _body,
        grid=(NH,),
        in_specs=[
            pl.BlockSpec((1, 1, W), lambda h: (h, 0, 0),
                         memory_space=pltpu.SMEM),
        ],
        out_specs=pl.BlockSpec(memory_space=pl.ANY),
        out_shape=jax.ShapeDtypeStruct((NH, S, S), jnp.float32),
        scratch_shapes=[
            pltpu.VMEM((2, B, MLEN), jnp.float32),
            pltpu.SemaphoreType.DMA((2,)),
        ],
    )(relative_bias.reshape(NH, 1, W))
    return out
